# Initial kernel scaffold; baseline (speedup 1.0000x reference)
import jax
import jax.numpy as jnp
from jax.experimental import pallas as pl


def kernel(x, slices, W):
    raise NotImplementedError("write your pallas kernel here")


# speedup vs baseline: 7.0892x; 7.0892x over previous
import jax
import jax.numpy as jnp
from jax import lax
from jax.experimental import pallas as pl
from jax.experimental.pallas import tpu as pltpu
from jax.experimental.pallas import tpu_sc as plsc

N_TOK = 32768
N_SEG = 16
N_SUB = 16
CHUNK = N_TOK // N_SUB
NV = CHUNK // 16


def _sc_body(x_hbm, coef_hbm, bnd_hbm, out_hbm,
             xv, yv, segv, coefv, bndv, pbuf, allp, invv, shared):
    cid = lax.axis_index("c")
    sid = lax.axis_index("s")

    @pl.when(cid == 0)
    def _():
        base = sid * CHUNK
        pltpu.sync_copy(x_hbm.at[pl.ds(base, CHUNK)], xv)
        pltpu.sync_copy(coef_hbm, coefv)
        pltpu.sync_copy(bnd_hbm, bndv)

        iota = lax.broadcasted_iota(jnp.int32, (16,), 0)
        gbase = jnp.left_shift(jnp.right_shift(iota, 2), 2)
        wrows = [coefv[pl.ds(16 * k, 16)] for k in range(4)]
        bvecs = [plsc.load_gather(bndv, [jnp.full((16,), s, jnp.int32)])
                 for s in range(1, N_SEG)]

        zv = jnp.zeros((16,), jnp.float32)
        for s in range(N_SEG):
            pbuf[pl.ds(16 * s, 16)] = zv

        def sA(t, carry):
            off = t * 16
            y = wrows[0] * plsc.load_gather(xv, [off + gbase])
            for k in range(1, 4):
                y = y + wrows[k] * plsc.load_gather(xv, [off + gbase + k])
            yv[pl.ds(off, 16)] = y
            p = base + off + iota
            seg = jnp.zeros((16,), jnp.int32)
            for b in bvecs:
                seg = seg + jnp.where(p >= b, 1, 0)
            segv[pl.ds(off, 16)] = seg
            plsc.addupdate_scatter(pbuf, [seg * 16 + iota], jnp.abs(y))
            return carry

        lax.fori_loop(0, NV, sA, 0)

        pltpu.sync_copy(pbuf, shared.at[pl.ds(sid * 16 * N_SEG, 16 * N_SEG)])
        plsc.subcore_barrier()
        pltpu.sync_copy(shared, allp)

        inv = zv
        for s in range(N_SEG):
            tv = allp[pl.ds(16 * s, 16)]
            for w in range(1, N_SUB):
                tv = tv + allp[pl.ds(16 * N_SEG * w + 16 * s, 16)]
            tot = jnp.sum(tv)
            inv = inv + jnp.where(iota == s,
                                  jnp.full((16,), 1.0, jnp.float32) / tot, zv)
        invv[...] = inv

        def sB(t, carry):
            off = t * 16
            seg = segv[pl.ds(off, 16)]
            iv = plsc.load_gather(invv, [seg])
            yv[pl.ds(off, 16)] = yv[pl.ds(off, 16)] * iv
            return carry

        lax.fori_loop(0, NV, sB, 0)
        pltpu.sync_copy(yv, out_hbm.at[pl.ds(base, CHUNK)])


def _coef_table(W):
    lane = jnp.arange(16) % 4
    rows = [W[lane, k] for k in range(4)]
    return jnp.concatenate(rows).astype(jnp.float32)


@jax.jit
def kernel(x, slices, W):
    bnd = jnp.zeros((32,), jnp.int32).at[:N_SEG + 1].set(
        slices.astype(jnp.int32))
    coef = _coef_table(W)

    mesh = plsc.VectorSubcoreMesh(
        core_axis_name="c", subcore_axis_name="s", num_cores=2)
    kfn = pl.kernel(
        _sc_body,
        out_type=jax.ShapeDtypeStruct((N_TOK,), jnp.float32),
        mesh=mesh,
        scratch_types=[
            pltpu.VMEM((CHUNK,), jnp.float32),
            pltpu.VMEM((CHUNK,), jnp.float32),
            pltpu.VMEM((CHUNK,), jnp.int32),
            pltpu.VMEM((64,), jnp.float32),
            pltpu.VMEM((32,), jnp.int32),
            pltpu.VMEM((16 * N_SEG,), jnp.float32),
            pltpu.VMEM((N_SUB * 16 * N_SEG,), jnp.float32),
            pltpu.VMEM((16,), jnp.float32),
            pltpu.VMEM_SHARED((N_SUB * 16 * N_SEG,), jnp.float32),
        ],
        compiler_params=pltpu.CompilerParams(needs_layout_passes=False),
    )
    return kfn(x, coef, bnd)


# speedup vs baseline: 7.5534x; 1.0655x over previous
import jax
import jax.numpy as jnp
from jax import lax
from jax.experimental import pallas as pl
from jax.experimental.pallas import tpu as pltpu
from jax.experimental.pallas import tpu_sc as plsc

N_TOK = 32768
N_SEG = 16
N_SUB = 16
CHUNK = N_TOK // N_SUB
NV = CHUNK // 16


def _sc_body(x_hbm, coef_hbm, bnd_hbm, out_hbm,
             xv, yv, segv, coefv, bndv, pbuf, allp, invv, shared):
    cid = lax.axis_index("c")
    sid = lax.axis_index("s")

    @pl.when(cid == 0)
    def _():
        base = sid * CHUNK
        pltpu.sync_copy(x_hbm.at[pl.ds(base, CHUNK)], xv)
        pltpu.sync_copy(coef_hbm, coefv)
        pltpu.sync_copy(bnd_hbm, bndv)

        iota = lax.broadcasted_iota(jnp.int32, (16,), 0)
        gbase = jnp.left_shift(jnp.right_shift(iota, 2), 2)
        wrows = [coefv[pl.ds(16 * k, 16)] for k in range(4)]
        bvecs = [plsc.load_gather(bndv, [jnp.full((16,), s, jnp.int32)])
                 for s in range(1, N_SEG)]

        zv = jnp.zeros((16,), jnp.float32)
        for s in range(N_SEG):
            pbuf[pl.ds(16 * s, 16)] = zv

        def sA(t, carry):
            off = t * 16
            y = wrows[0] * plsc.load_gather(xv, [off + gbase])
            for k in range(1, 4):
                y = y + wrows[k] * plsc.load_gather(xv, [off + gbase + k])
            yv[pl.ds(off, 16)] = y
            p = base + off + iota
            seg = jnp.zeros((16,), jnp.int32)
            for b in bvecs:
                seg = seg + jnp.where(p >= b, 1, 0)
            segv[pl.ds(off, 16)] = seg
            plsc.addupdate_scatter(pbuf, [seg * 16 + iota], jnp.abs(y))
            return carry

        lax.fori_loop(0, NV, sA, 0)

        pltpu.sync_copy(pbuf, shared.at[pl.ds(sid * 16 * N_SEG, 16 * N_SEG)])
        plsc.subcore_barrier()
        pltpu.sync_copy(shared, allp)

        inv = zv
        for s in range(N_SEG):
            tv = allp[pl.ds(16 * s, 16)]
            for w in range(1, N_SUB):
                tv = tv + allp[pl.ds(16 * N_SEG * w + 16 * s, 16)]
            tot = jnp.sum(tv)
            inv = inv + jnp.where(iota == s,
                                  jnp.full((16,), 1.0, jnp.float32) / tot, zv)
        invv[...] = inv

        def sB(t, carry):
            off = t * 16
            seg = segv[pl.ds(off, 16)]
            iv = plsc.load_gather(invv, [seg])
            yv[pl.ds(off, 16)] = yv[pl.ds(off, 16)] * iv
            return carry

        lax.fori_loop(0, NV, sB, 0)
        pltpu.sync_copy(yv, out_hbm.at[pl.ds(base, CHUNK)])


def _coef_table(W):
    lane = jnp.arange(16) % 4
    rows = [W[lane, k] for k in range(4)]
    return jnp.concatenate(rows).astype(jnp.float32)


@jax.jit
def kernel(x, slices, W):
    bnd = jnp.zeros((32,), jnp.int32).at[:N_SEG + 1].set(
        slices.astype(jnp.int32))
    coef = _coef_table(W)

    mesh = plsc.VectorSubcoreMesh(
        core_axis_name="c", subcore_axis_name="s", num_cores=1)
    kfn = pl.kernel(
        _sc_body,
        out_type=jax.ShapeDtypeStruct((N_TOK,), jnp.float32),
        mesh=mesh,
        scratch_types=[
            pltpu.VMEM((CHUNK,), jnp.float32),
            pltpu.VMEM((CHUNK,), jnp.float32),
            pltpu.VMEM((CHUNK,), jnp.int32),
            pltpu.VMEM((64,), jnp.float32),
            pltpu.VMEM((32,), jnp.int32),
            pltpu.VMEM((16 * N_SEG,), jnp.float32),
            pltpu.VMEM((N_SUB * 16 * N_SEG,), jnp.float32),
            pltpu.VMEM((16,), jnp.float32),
            pltpu.VMEM_SHARED((N_SUB * 16 * N_SEG,), jnp.float32),
        ],
        compiler_params=pltpu.CompilerParams(needs_layout_passes=False),
    )
    return kfn(x, coef, bnd)
